# Initial kernel scaffold; baseline (speedup 1.0000x reference)
import jax
import jax.numpy as jnp
from jax.experimental import pallas as pl


def kernel(input_ids, past_length, token_type_ids, word_embeddings, position_embeddings, token_type_embeddings, ln_weight, ln_bias):
    raise NotImplementedError("write your pallas kernel here")


# speedup vs baseline: 2.4498x; 2.4498x over previous
import functools

import jax
import jax.numpy as jnp
from jax import lax
from jax.experimental import pallas as pl
from jax.experimental.pallas import tpu as pltpu
from jax.experimental.pallas import tpu_sc as plsc

VOCAB = 21128
MAX_POS = 512
N_TYPES = 2
D = 768
B = 128
S = 512
EPS = 1e-12

NLANE = 16
NCORE = 2
NSUB = 16
NW = NCORE * NSUB
N_TOK = B * S
TPW = N_TOK // NW
G = 16
NCH = TPW // G
NBUF = 4
NPB = 2
DJ = D // NLANE
IW = TPW // 128


def _rsqrt_vec(x):
    i = lax.bitcast_convert_type(x, jnp.int32)
    i = jnp.int32(0x5F3759DF) - (i >> 1)
    y = lax.bitcast_convert_type(i, jnp.float32)
    for _ in range(4):
        y = y * (1.5 - 0.5 * x * y * y)
    return y


def _allsum(x):
    lane = lax.iota(jnp.int32, NLANE)
    for sft in (1, 2, 4, 8):
        x = x + x.at[lane ^ sft].get(mode="promise_in_bounds",
                                     unique_indices=True)
    return x


def _bcast_lane(x, r):
    idx = jnp.full((NLANE,), r, dtype=jnp.int32)
    return x.at[idx].get(mode="promise_in_bounds")


def _ln_chunk(wb, pb, ttv, wv, bv, dv):
    zero = jnp.zeros((NLANE,), jnp.float32)
    ttr = [_bcast_lane(ttv, r) for r in range(G)]

    def p1_body(j, carry):
        acc, acc2 = carry
        sl = pl.ds(j * NLANE, NLANE)
        dj = dv[sl]
        acc_n = []
        acc2_n = []
        for r in range(G):
            v = wb[r, sl] + pb[r, sl] + ttr[r] * dj
            wb[r, sl] = v
            acc_n.append(acc[r] + v)
            acc2_n.append(acc2[r] + v * v)
        return tuple(acc_n), tuple(acc2_n)

    acc, acc2 = lax.fori_loop(0, DJ, p1_body,
                              ((zero,) * G, (zero,) * G))

    mvec = []
    ivec = []
    for r in range(G):
        mean = _allsum(acc[r]) * (1.0 / D)
        var = _allsum(acc2[r]) * (1.0 / D) - mean * mean
        mvec.append(mean)
        ivec.append(_rsqrt_vec(var + EPS))

    def p2_body(j, carry):
        sl = pl.ds(j * NLANE, NLANE)
        wj = wv[sl]
        bj = bv[sl]
        for r in range(G):
            v = wb[r, sl]
            wb[r, sl] = (v - mvec[r]) * ivec[r] * wj + bj
        return carry

    lax.fori_loop(0, DJ, p2_body, 0)


def _sc_embed_ln(widx_hbm, ttf_hbm, wtab_hbm, ptab_hbm, dt_hbm,
                 lnw_hbm, lnb_hbm,
                 out_hbm, widx_v, ttb, wbufs, pbufs, wv, bv, dv, p_sp,
                 semw, sempt, semo):
    wid = lax.axis_index("s") * NCORE + lax.axis_index("c")
    tok0 = wid * TPW

    @pl.when(lax.axis_index("s") == 0)
    def _():
        pltpu.sync_copy(ptab_hbm, p_sp)
    plsc.subcore_barrier()

    pltpu.sync_copy(widx_hbm.at[pl.ds(wid * IW, IW)], widx_v)
    pltpu.sync_copy(ttf_hbm.at[pl.ds(wid * IW, IW)], ttb)
    pltpu.sync_copy(lnw_hbm, wv)
    pltpu.sync_copy(lnb_hbm, bv)
    pltpu.sync_copy(dt_hbm, dv)

    def islice(c):
        c16 = c * G
        return widx_v.at[c16 // 128, pl.ds(c16 % 128, G)]

    def start_w(c, k):
        pltpu.async_copy(wtab_hbm.at[islice(c)], wbufs[k], semw.at[k])

    def wait_w(c, k):
        pltpu.make_async_copy(wtab_hbm.at[islice(c)], wbufs[k],
                              semw.at[k]).wait()

    def start_pt(c, k):
        pltpu.async_copy(p_sp.at[pl.ds((c * G) % MAX_POS, G)], pbufs[k],
                         sempt.at[k])

    def wait_pt(c, k):
        pltpu.make_async_copy(p_sp.at[pl.ds((c * G) % MAX_POS, G)],
                              pbufs[k], sempt.at[k]).wait()

    def start_out(c, k):
        pltpu.async_copy(wbufs[k], out_hbm.at[pl.ds(tok0 + c * G, G)],
                         semo.at[k])

    def wait_out(c, k):
        pltpu.make_async_copy(wbufs[k], out_hbm.at[pl.ds(tok0 + c * G, G)],
                              semo.at[k]).wait()

    start_w(0, 0)
    start_w(1, 1)
    for k in range(NPB):
        start_pt(k, k)

    @pl.loop(0, NCH // NBUF)
    def _outer(i):
        for b in range(NBUF):
            c = i * NBUF + b
            k = b
            kp = b % NPB
            c16 = c * G
            ttv = ttb[c16 // 128, pl.ds(c16 % 128, NLANE)]
            wait_w(c, k)
            wait_pt(c, kp)
            _ln_chunk(wbufs[k], pbufs[kp], ttv, wv, bv, dv)
            cond = (i < NCH // NBUF - 1) if b >= NBUF - NPB else True
            @pl.when(cond)
            def _():
                start_pt(c + NPB, kp)
            kn = (k + 2) % NBUF
            if b < 2:
                @pl.when(i >= 1)
                def _():
                    wait_out(c - 2, kn)
                start_w(c + 2, kn)
            else:
                @pl.when(i < NCH // NBUF - 1)
                def _():
                    wait_out(c - 2, kn)
                    start_w(c + 2, kn)
            start_out(c, k)

    for c in range(NCH - NBUF, NCH):
        wait_out(c, c % NBUF)


def kernel(input_ids, past_length, token_type_ids, word_embeddings,
           position_embeddings, token_type_embeddings, ln_weight, ln_bias):
    ids = input_ids.astype(jnp.int32)
    tts = token_type_ids.astype(jnp.int32)
    seq = ids.shape[-1]

    pos = jnp.arange(seq, dtype=jnp.int32) + jnp.asarray(past_length, jnp.int32)
    pos = jnp.clip(pos, 0, MAX_POS - 1)
    widx = jnp.clip(ids, 0, VOCAB - 1).reshape(N_TOK // 128, 128)
    ttf = tts.astype(jnp.float32).reshape(N_TOK // 128, 128)
    p_table = (jnp.take(position_embeddings, pos, axis=0)
               + token_type_embeddings[0][None, :])
    dt_vec = token_type_embeddings[1] - token_type_embeddings[0]

    mesh = plsc.VectorSubcoreMesh(core_axis_name="c", subcore_axis_name="s",
                                  num_cores=NCORE, num_subcores=NSUB)
    run = functools.partial(
        pl.kernel,
        out_type=jax.ShapeDtypeStruct((N_TOK, D), jnp.float32),
        mesh=mesh,
        scratch_types=[
            pltpu.VMEM((IW, 128), jnp.int32),
            pltpu.VMEM((IW, 128), jnp.float32),
            [pltpu.VMEM((G, D), jnp.float32) for _ in range(NBUF)],
            [pltpu.VMEM((G, D), jnp.float32) for _ in range(NPB)],
            pltpu.VMEM((D,), jnp.float32),
            pltpu.VMEM((D,), jnp.float32),
            pltpu.VMEM((D,), jnp.float32),
            pltpu.VMEM_SHARED((MAX_POS, D), jnp.float32),
            pltpu.SemaphoreType.DMA((NBUF,)),
            pltpu.SemaphoreType.DMA((NPB,)),
            pltpu.SemaphoreType.DMA((NBUF,)),
        ],
    )(_sc_embed_ln)
    out = run(widx, ttf, word_embeddings, p_table, dt_vec,
              ln_weight.astype(jnp.float32), ln_bias.astype(jnp.float32))
    return out.reshape(input_ids.shape[0], seq, D)
